# baseline (speedup 1.0000x reference)
import functools

import jax
import jax.numpy as jnp
from jax import lax
from jax.experimental import pallas as pl
from jax.experimental.pallas import tpu as pltpu, tpu_sc as plsc

_B = 16
_NC = 6
_CW = 73728
_NISS = 8
_IMG_CHUNKS_PER_CORE = 288
_NGROUPS = _IMG_CHUNKS_PER_CORE // _NISS


def _sc_body(cam_hbm, imgs_hbm, imgs_out,
             cam_v, zstage, slots, zeros, sem_cam, sem_z, sem_in, sem_out):
    cid = lax.axis_index("c")
    sid = lax.axis_index("s")

    pltpu.make_async_copy(cam_hbm, cam_v.at[pl.ds(0, 16)], sem_cam).start()

    @pl.when(sid == 0)
    def _init_zeros():
        def _zf(i, _):
            for u in range(8):
                zstage[pl.ds(i * 128 + u * 16, 16)] = jnp.zeros((16,), jnp.float32)
            return 0
        lax.fori_loop(0, _CW // 128, _zf, 0)
        pltpu.make_async_copy(zstage, zeros, sem_z).start()
        pltpu.make_async_copy(zstage, zeros, sem_z).wait()

    pltpu.make_async_copy(cam_hbm, cam_v.at[pl.ds(0, 16)], sem_cam).wait()
    plsc.subcore_barrier()

    @pl.when(sid < _NISS)
    def _issue():
        slot = slots.at[sid]

        def _group(g, _):
            ch = g * _NISS + sid
            idx = cid * _IMG_CHUNKS_PER_CORE + ch
            row = idx // 6
            keep = cam_v[pl.ds(row // _NC, 16)][0] != row % _NC

            @pl.when(keep)
            def _copy():
                pltpu.make_async_copy(imgs_hbm.at[idx], slot, sem_in).start()
                pltpu.make_async_copy(imgs_hbm.at[idx], slot, sem_in).wait()
                pltpu.make_async_copy(slot, imgs_out.at[idx], sem_out).start()
                pltpu.make_async_copy(slot, imgs_out.at[idx], sem_out).wait()

            @pl.when(jnp.logical_not(keep))
            def _blank():
                pltpu.make_async_copy(zeros, imgs_out.at[idx], sem_out).start()
                pltpu.make_async_copy(zeros, imgs_out.at[idx], sem_out).wait()

            return 0

        lax.fori_loop(0, _NGROUPS, _group, 0)


def _tc_body(cam_ref, masks_ref, masks_out_ref):
    p = pl.program_id(0)
    keep = jnp.where(cam_ref[p // _NC] == p % _NC, 0.0, 1.0).astype(jnp.float32)
    masks_out_ref[...] = masks_ref[...] * keep


def kernel(imgs, grids, masks, cam_choice):
    B, NC, C, H, W = imgs.shape
    cam32 = cam_choice.astype(jnp.int32)
    imgs3 = imgs.reshape(B * NC * 6, _CW)
    masks2 = masks.reshape(B * NC, 128, 1152)

    mesh = plsc.VectorSubcoreMesh(core_axis_name="c", subcore_axis_name="s")
    sc = functools.partial(
        pl.kernel,
        out_type=jax.ShapeDtypeStruct(imgs3.shape, imgs3.dtype),
        mesh=mesh,
        scratch_types=[
            pltpu.VMEM((32,), jnp.int32),
            pltpu.VMEM((_CW,), jnp.float32),
            pltpu.MemorySpace.VMEM_SHARED((_NISS, _CW), jnp.float32),
            pltpu.MemorySpace.VMEM_SHARED((_CW,), jnp.float32),
            pltpu.SemaphoreType.DMA,
            pltpu.SemaphoreType.DMA,
            pltpu.SemaphoreType.DMA,
            pltpu.SemaphoreType.DMA,
        ],
    )(_sc_body)

    imgs_out = sc(cam32, imgs3)

    masks_out = pl.pallas_call(
        _tc_body,
        grid_spec=pltpu.PrefetchScalarGridSpec(
            num_scalar_prefetch=1,
            grid=(B * NC,),
            in_specs=[pl.BlockSpec((1, 128, 1152), lambda p, cam: (p, 0, 0))],
            out_specs=pl.BlockSpec((1, 128, 1152), lambda p, cam: (p, 0, 0)),
        ),
        out_shape=jax.ShapeDtypeStruct(masks2.shape, masks2.dtype),
    )(cam32, masks2)

    return (imgs_out.reshape(imgs.shape), grids, masks_out.reshape(masks.shape))
